# baseline (speedup 1.0000x reference)
import jax
import jax.numpy as jnp
from jax import lax
from jax.experimental import pallas as pl
from jax.experimental.pallas import tpu as pltpu

T = 2048
NHID = 1024
NCLS = 100
CHUNK = 1000
P = 20
G = 5
NSTEP = NCLS // G


def _tcg_body(idx_ref, x_ref, xany_ref, Wc_ref, bc_ref, Ww_ref, bw_ref,
              pclass_ref, pwords_ref, rows_ref, sems):
    c = pl.program_id(0)

    def fetch_rows(step, b):
        for j in range(G):
            for i in range(P):
                pltpu.make_async_copy(
                    xany_ref.at[pl.ds(idx_ref[step * G + j, i], 1)],
                    rows_ref.at[b, pl.ds(j * P + i, 1)],
                    sems.at[b]).start()

    def wait_rows(step, b):
        for j in range(G):
            for i in range(P):
                pltpu.make_async_copy(
                    xany_ref.at[pl.ds(idx_ref[step * G + j, i], 1)],
                    rows_ref.at[b, pl.ds(j * P + i, 1)],
                    sems.at[b]).wait()

    @pl.when(c == 0)
    def _():
        fetch_rows(0, 0)
        pc = lax.dot_general(x_ref[...], Wc_ref[...],
                             (((1,), (1,)), ((), ())),
                             preferred_element_type=jnp.float32)
        pclass_ref[...] = pc + bc_ref[...]

    @pl.when(c + 1 < pl.num_programs(0))
    def _():
        fetch_rows(c + 1, (c + 1) % 2)

    wait_rows(c, c % 2)
    for j in range(G):
        d = rows_ref[c % 2, j * P:(j + 1) * P, :]
        w = Ww_ref[j]
        pw = lax.dot_general(d, w, (((1,), (1,)), ((), ())),
                             preferred_element_type=jnp.float32)
        pwords_ref[j] = pw + bw_ref[j]


def _decode(idx, x, Wc, bc2, Ww, bw3):
    grid_spec = pltpu.PrefetchScalarGridSpec(
        num_scalar_prefetch=1,
        grid=(NSTEP,),
        in_specs=[
            pl.BlockSpec((T, NHID), lambda c, i_: (0, 0)),
            pl.BlockSpec(memory_space=pl.ANY),
            pl.BlockSpec((NCLS, NHID), lambda c, i_: (0, 0)),
            pl.BlockSpec((1, NCLS), lambda c, i_: (0, 0)),
            pl.BlockSpec((G, CHUNK, NHID), lambda c, i_: (c, 0, 0)),
            pl.BlockSpec((G, 1, CHUNK), lambda c, i_: (c, 0, 0)),
        ],
        out_specs=[
            pl.BlockSpec((T, NCLS), lambda c, i_: (0, 0)),
            pl.BlockSpec((G, P, CHUNK), lambda c, i_: (c, 0, 0)),
        ],
        scratch_shapes=[
            pltpu.VMEM((2, G * P, NHID), jnp.float32),
            pltpu.SemaphoreType.DMA((2,)),
        ],
    )
    return pl.pallas_call(
        _tcg_body,
        grid_spec=grid_spec,
        out_shape=[
            jax.ShapeDtypeStruct((T, NCLS), jnp.float32),
            jax.ShapeDtypeStruct((NCLS, P, CHUNK), jnp.float32),
        ],
    )(idx, x, x, Wc, bc2, Ww, bw3)


def kernel(x, within_batch_idx, Wc, bc, Ww, bw):
    idx32 = within_batch_idx.astype(jnp.int32)
    p_class, p_words = _decode(idx32, x, Wc, bc.reshape(1, NCLS),
                               Ww, bw.reshape(NCLS, 1, CHUNK))
    return (p_class, p_words)


# speedup vs baseline: 1.0207x; 1.0207x over previous
import jax
import jax.numpy as jnp
from jax import lax
from jax.experimental import pallas as pl
from jax.experimental.pallas import tpu as pltpu

T = 2048
NHID = 1024
NCLS = 100
CHUNK = 1000
P = 20
G = 2
NSTEP = NCLS // G


def _tcg_body(idx_ref, x_ref, xany_ref, Wc_ref, bc_ref, Ww_ref, bw_ref,
              pclass_ref, pwords_ref, rows_ref, sems):
    c = pl.program_id(0)

    def fetch_rows(step, b):
        for j in range(G):
            for i in range(P):
                pltpu.make_async_copy(
                    xany_ref.at[pl.ds(idx_ref[step * G + j, i], 1)],
                    rows_ref.at[b, pl.ds(j * P + i, 1)],
                    sems.at[b]).start()

    def wait_rows(step, b):
        for j in range(G):
            for i in range(P):
                pltpu.make_async_copy(
                    xany_ref.at[pl.ds(idx_ref[step * G + j, i], 1)],
                    rows_ref.at[b, pl.ds(j * P + i, 1)],
                    sems.at[b]).wait()

    @pl.when(c == 0)
    def _():
        fetch_rows(0, 0)
        pc = lax.dot_general(x_ref[...], Wc_ref[...],
                             (((1,), (1,)), ((), ())),
                             preferred_element_type=jnp.float32)
        pclass_ref[...] = pc + bc_ref[...]

    @pl.when(c + 1 < pl.num_programs(0))
    def _():
        fetch_rows(c + 1, (c + 1) % 2)

    wait_rows(c, c % 2)
    for j in range(G):
        d = rows_ref[c % 2, j * P:(j + 1) * P, :]
        w = Ww_ref[j]
        pw = lax.dot_general(d, w, (((1,), (1,)), ((), ())),
                             preferred_element_type=jnp.float32)
        pwords_ref[j] = pw + bw_ref[j]


def _decode(idx, x, Wc, bc2, Ww, bw3):
    grid_spec = pltpu.PrefetchScalarGridSpec(
        num_scalar_prefetch=1,
        grid=(NSTEP,),
        in_specs=[
            pl.BlockSpec((T, NHID), lambda c, i_: (0, 0)),
            pl.BlockSpec(memory_space=pl.ANY),
            pl.BlockSpec((NCLS, NHID), lambda c, i_: (0, 0)),
            pl.BlockSpec((1, NCLS), lambda c, i_: (0, 0)),
            pl.BlockSpec((G, CHUNK, NHID), lambda c, i_: (c, 0, 0)),
            pl.BlockSpec((G, 1, CHUNK), lambda c, i_: (c, 0, 0)),
        ],
        out_specs=[
            pl.BlockSpec((T, NCLS), lambda c, i_: (0, 0)),
            pl.BlockSpec((G, P, CHUNK), lambda c, i_: (c, 0, 0)),
        ],
        scratch_shapes=[
            pltpu.VMEM((2, G * P, NHID), jnp.float32),
            pltpu.SemaphoreType.DMA((2,)),
        ],
    )
    return pl.pallas_call(
        _tcg_body,
        grid_spec=grid_spec,
        out_shape=[
            jax.ShapeDtypeStruct((T, NCLS), jnp.float32),
            jax.ShapeDtypeStruct((NCLS, P, CHUNK), jnp.float32),
        ],
    )(idx, x, x, Wc, bc2, Ww, bw3)


def kernel(x, within_batch_idx, Wc, bc, Ww, bw):
    idx32 = within_batch_idx.astype(jnp.int32)
    p_class, p_words = _decode(idx32, x, Wc, bc.reshape(1, NCLS),
                               Ww, bw.reshape(NCLS, 1, CHUNK))
    return (p_class, p_words)
